# Initial kernel scaffold; baseline (speedup 1.0000x reference)
import functools

import jax
import jax.numpy as jnp
from jax import lax
from jax.experimental import pallas as pl
from jax.experimental.pallas import tpu as pltpu
from jax.experimental.pallas import tpu_sc as plsc

B = 16384
D = 64
K = 20
NC = 2
NS = 16
NW = NC * NS
EPW = B // NW
CH = 32
NCH = EPW // CH
NEG_IDX_ROWS = B * K // 128


def _sc_body(u_tab, v_tab, u_idx_h, v_idx_h, neg_idx_h,
             pos_h, neg_h,
             u_idx, v_idx, neg_idx, u_rows, v_rows, neg_rows,
             pos_out, neg_out, sem):
    cid = lax.axis_index("c")
    sid = lax.axis_index("s")
    wid = sid * NC + cid

    pltpu.sync_copy(u_idx_h.at[pl.ds(wid * NCH, NCH)], u_idx)
    pltpu.sync_copy(v_idx_h.at[pl.ds(wid * NCH, NCH)], v_idx)
    pltpu.sync_copy(neg_idx_h.at[pl.ds(wid * (NCH * 5), NCH * 5)], neg_idx)

    iota = lax.iota(jnp.int32, 16)
    zero = jnp.zeros((16,), jnp.float32)

    @pl.loop(0, NCH)
    def _chunk(c):
        cps = [
            pltpu.async_copy(u_tab.at[u_idx.at[c]], u_rows, sem),
            pltpu.async_copy(v_tab.at[v_idx.at[c]], v_rows, sem),
        ]
        for j in range(5):
            cps.append(pltpu.async_copy(
                v_tab.at[neg_idx.at[c * 5 + j]],
                neg_rows.at[pl.ds(j * 128, 128)], sem))
        for cp in cps:
            cp.wait()

        for g in range(CH // 16):
            e = g * 16 + iota
            e20 = e * K

            def dbody(dd, accs, e=e, e20=e20):
                col = lax.broadcast(dd, (16,))
                ut = plsc.load_gather(u_rows, [e, col])
                vt = plsc.load_gather(v_rows, [e, col])
                new = [accs[0] + ut * vt]
                for k in range(K):
                    nt = plsc.load_gather(neg_rows, [e20 + k, col])
                    new.append(accs[k + 1] + ut * nt)
                return tuple(new)

            accs = lax.fori_loop(0, D, dbody, (zero,) * (K + 1))

            base = c * CH + g * 16
            plsc.store_scatter(pos_out, [base + iota], accs[0])
            ewk = base * K + iota * K
            for k in range(K):
                plsc.store_scatter(neg_out, [ewk + k], accs[k + 1])

    pltpu.sync_copy(pos_out, pos_h.at[pl.ds(wid * EPW, EPW)])
    pltpu.sync_copy(neg_out, neg_h.at[pl.ds(wid * EPW * K, EPW * K)])


_sc_scores = functools.partial(
    pl.kernel,
    out_type=(jax.ShapeDtypeStruct((B,), jnp.float32),
              jax.ShapeDtypeStruct((B * K,), jnp.float32)),
    mesh=plsc.VectorSubcoreMesh(core_axis_name="c", subcore_axis_name="s",
                                num_cores=NC, num_subcores=NS),
    scratch_types=[
        pltpu.VMEM((NCH, CH), jnp.int32),
        pltpu.VMEM((NCH, CH), jnp.int32),
        pltpu.VMEM((NCH * 5, 128), jnp.int32),
        pltpu.VMEM((CH, D), jnp.float32),
        pltpu.VMEM((CH, D), jnp.float32),
        pltpu.VMEM((CH * K, D), jnp.float32),
        pltpu.VMEM((EPW,), jnp.float32),
        pltpu.VMEM((EPW * K,), jnp.float32),
        pltpu.SemaphoreType.DMA,
    ],
)(_sc_body)


def _loss_body(pos_ref, neg_ref, out_ref):
    p = pos_ref[...]
    n = -neg_ref[...]
    ls_p = jnp.minimum(p, 0.0) - jnp.log1p(jnp.exp(-jnp.abs(p)))
    ls_n = jnp.minimum(n, 0.0) - jnp.log1p(jnp.exp(-jnp.abs(n)))
    loss = -(jnp.sum(ls_p) / B + jnp.sum(ls_n) / (B * K)) * 0.5
    out_ref[0, 0] = loss


_loss_call = pl.pallas_call(
    _loss_body,
    out_shape=jax.ShapeDtypeStruct((1, 1), jnp.float32),
    out_specs=pl.BlockSpec(memory_space=pltpu.SMEM),
)


def kernel(u_table, v_table, u, v, negative_v):
    u2 = u.reshape(NW * NCH, CH)
    v2 = v.reshape(NW * NCH, CH)
    neg2 = negative_v.reshape(NEG_IDX_ROWS, 128)
    pos, neg = _sc_scores(u_table, v_table, u2, v2, neg2)
    loss = _loss_call(pos.reshape(128, 128), neg.reshape(NEG_IDX_ROWS, 128))
    return loss[0, 0]


# speedup vs baseline: 4.0269x; 4.0269x over previous
import functools

import jax
import jax.numpy as jnp
from jax import lax
from jax.experimental import pallas as pl
from jax.experimental.pallas import tpu as pltpu
from jax.experimental.pallas import tpu_sc as plsc

B = 16384
D = 64
K = 20
NC = 2
NS = 16
NW = NC * NS
EPW = B // NW
CH = 32
NCH = EPW // CH
NEG_IDX_ROWS = B * K // 128


def _sc_body(u_tab, v_tab, u_idx_h, v_idx_h, neg_idx_h,
             pos_h, neg_h,
             u_idx, v_idx, neg_idx, u_rows, v_rows, neg_rows,
             pos_out, neg_out, sem):
    cid = lax.axis_index("c")
    sid = lax.axis_index("s")
    wid = sid * NC + cid

    pltpu.sync_copy(u_idx_h.at[pl.ds(wid * NCH, NCH)], u_idx)
    pltpu.sync_copy(v_idx_h.at[pl.ds(wid * NCH, NCH)], v_idx)
    pltpu.sync_copy(neg_idx_h.at[pl.ds(wid * (NCH * 5), NCH * 5)], neg_idx)

    iota = lax.iota(jnp.int32, 16)
    zero = jnp.zeros((16,), jnp.float32)

    @pl.loop(0, NCH)
    def _chunk(c):
        cps = [
            pltpu.async_copy(u_tab.at[u_idx.at[c]], u_rows, sem),
            pltpu.async_copy(v_tab.at[v_idx.at[c]], v_rows, sem),
        ]
        for j in range(5):
            cps.append(pltpu.async_copy(
                v_tab.at[neg_idx.at[c * 5 + j]],
                neg_rows.at[pl.ds(j * 128, 128)], sem))
        for cp in cps:
            cp.wait()

        for g in range(CH // 16):
            e = g * 16 + iota
            e20 = e * K

            def dbody(dd, accs, e=e, e20=e20):
                col = lax.broadcast(dd, (16,))
                ut = plsc.load_gather(u_rows, [e, col])
                vt = plsc.load_gather(v_rows, [e, col])
                new = [accs[0] + ut * vt]
                for k in range(K):
                    nt = plsc.load_gather(neg_rows, [e20 + k, col])
                    new.append(accs[k + 1] + ut * nt)
                return tuple(new)

            accs = lax.fori_loop(0, D, dbody, (zero,) * (K + 1))

            base = c * CH + g * 16
            plsc.store_scatter(pos_out, [base + iota], accs[0])
            ewk = base * K + iota * K
            for k in range(K):
                plsc.store_scatter(neg_out, [ewk + k], accs[k + 1])

    pltpu.sync_copy(pos_out, pos_h.at[pl.ds(wid * EPW, EPW)])
    pltpu.sync_copy(neg_out, neg_h.at[pl.ds(wid * EPW * K, EPW * K)])


_sc_scores = functools.partial(
    pl.kernel,
    out_type=(jax.ShapeDtypeStruct((B,), jnp.float32),
              jax.ShapeDtypeStruct((B * K,), jnp.float32)),
    mesh=plsc.VectorSubcoreMesh(core_axis_name="c", subcore_axis_name="s",
                                num_cores=NC, num_subcores=NS),
    scratch_types=[
        pltpu.VMEM((NCH, CH), jnp.int32),
        pltpu.VMEM((NCH, CH), jnp.int32),
        pltpu.VMEM((NCH * 5, 128), jnp.int32),
        pltpu.VMEM((CH, D), jnp.float32),
        pltpu.VMEM((CH, D), jnp.float32),
        pltpu.VMEM((CH * K, D), jnp.float32),
        pltpu.VMEM((EPW,), jnp.float32),
        pltpu.VMEM((EPW * K,), jnp.float32),
        pltpu.SemaphoreType.DMA,
    ],
    compiler_params=pltpu.CompilerParams(
        needs_layout_passes=False,
        use_tc_tiling_on_sc=False,
    ),
)(_sc_body)


def _loss_body(pos_ref, neg_ref, out_ref):
    p = pos_ref[...]
    n = -neg_ref[...]
    ls_p = jnp.minimum(p, 0.0) - jnp.log1p(jnp.exp(-jnp.abs(p)))
    ls_n = jnp.minimum(n, 0.0) - jnp.log1p(jnp.exp(-jnp.abs(n)))
    loss = -(jnp.sum(ls_p) / B + jnp.sum(ls_n) / (B * K)) * 0.5
    out_ref[0, 0] = loss


_loss_call = pl.pallas_call(
    _loss_body,
    out_shape=jax.ShapeDtypeStruct((1, 1), jnp.float32),
    out_specs=pl.BlockSpec(memory_space=pltpu.SMEM),
)


def kernel(u_table, v_table, u, v, negative_v):
    u2 = u.reshape(NW * NCH, CH)
    v2 = v.reshape(NW * NCH, CH)
    neg2 = negative_v.reshape(NEG_IDX_ROWS, 128)
    pos, neg = _sc_scores(u_table, v_table, u2, v2, neg2)
    loss = _loss_call(pos.reshape(128, 128), neg.reshape(NEG_IDX_ROWS, 128))
    return loss[0, 0]
